# baseline (speedup 1.0000x reference)
import functools

import jax
import jax.numpy as jnp
from jax import lax
from jax.experimental import pallas as pl
from jax.experimental.pallas import tpu as pltpu
from jax.experimental.pallas import tpu_sc as plsc

N_FIELDS = 26
VOCAB = 100000
EMB_DIM = 16
BATCH = 16384

NUM_WORKERS = 32
NUM_PLANES = N_FIELDS * EMB_DIM
PLANES_PER_W = NUM_PLANES // NUM_WORKERS
OUT_CHUNK = 4096
NCHUNK = BATCH // OUT_CHUNK


def _sc_lookup(x_t, tab_t):
    mesh = plsc.VectorSubcoreMesh(core_axis_name="c", subcore_axis_name="s")

    @functools.partial(
        pl.kernel,
        out_type=jax.ShapeDtypeStruct((N_FIELDS, EMB_DIM, BATCH), jnp.float32),
        mesh=mesh,
        scratch_types=[
            pltpu.VMEM((VOCAB,), jnp.float32),
            pltpu.VMEM((BATCH,), jnp.int32),
            pltpu.VMEM((OUT_CHUNK,), jnp.float32),
            pltpu.VMEM((OUT_CHUNK,), jnp.float32),
            pltpu.SemaphoreType.DMA,
            pltpu.SemaphoreType.DMA,
            pltpu.SemaphoreType.DMA,
            pltpu.SemaphoreType.DMA,
        ],
        compiler_params=pltpu.CompilerParams(needs_layout_passes=False),
    )
    def run(x_hbm, tab_hbm, out_hbm, plane_v, idx_v, oc0, oc1, sp, si, so0, so1):
        wid = lax.axis_index("s") * 2 + lax.axis_index("c")
        p0 = wid * PLANES_PER_W
        ocs = (oc0, oc1)
        sos = (so0, so1)

        def do_plane(p, first_out):
            f = p // EMB_DIM
            d = p % EMB_DIM
            pltpu.async_copy(tab_hbm.at[f, d], plane_v, sp)
            pltpu.make_async_copy(tab_hbm.at[f, d], plane_v, sp).wait()

            for h in range(NCHUNK):
                ob = ocs[h % 2]
                sem = sos[h % 2]

                def drain(ob=ob, sem=sem):
                    pltpu.make_async_copy(
                        ob, out_hbm.at[0, 0, pl.ds(0, OUT_CHUNK)], sem).wait()

                if h < 2:
                    pl.when(jnp.logical_not(first_out))(drain)
                else:
                    drain()

                @plsc.parallel_loop(0, OUT_CHUNK // 16, unroll=8)
                def _(i, h=h, ob=ob):
                    vidx = idx_v[pl.ds(h * OUT_CHUNK + i * 16, 16)]
                    ob[pl.ds(i * 16, 16)] = plsc.load_gather(plane_v, [vidx])

                pltpu.async_copy(
                    ob, out_hbm.at[f, d, pl.ds(h * OUT_CHUNK, OUT_CHUNK)], sem)
            return jnp.bool_(False)

        f0 = p0 // EMB_DIM
        k_split = jnp.minimum(PLANES_PER_W, (f0 + 1) * EMB_DIM - p0)

        spin = lax.fori_loop(
            0, (wid % 4) * 700, lambda i, a: a * 3 + i, wid,
            unroll=False)
        oc0[pl.ds(0, 16)] = jax.lax.broadcast(spin, (16,)).astype(jnp.float32)

        pltpu.sync_copy(x_hbm.at[f0], idx_v)
        first = lax.fori_loop(
            0, k_split, lambda k, fo: do_plane(p0 + k, fo), jnp.bool_(True))

        @pl.when(k_split < PLANES_PER_W)
        def _():
            pltpu.sync_copy(x_hbm.at[f0 + 1], idx_v)
            lax.fori_loop(k_split, PLANES_PER_W,
                          lambda k, fo: do_plane(p0 + k, fo), first)

        pltpu.make_async_copy(
            oc0, out_hbm.at[0, 0, pl.ds(0, OUT_CHUNK)], so0).wait()
        pltpu.make_async_copy(
            oc1, out_hbm.at[0, 0, pl.ds(0, OUT_CHUNK)], so1).wait()

    return run(x_t, tab_t)


def kernel(x, tables):
    tab_t = tables.transpose(0, 2, 1)
    x_t = x.T
    out_t = _sc_lookup(x_t, tab_t)
    return out_t.transpose(2, 0, 1)
